# baseline (speedup 1.0000x reference)
import functools

import numpy as np
import jax
import jax.numpy as jnp
from jax import lax
from jax.experimental import pallas as pl
from jax.experimental.pallas import tpu as pltpu

_NBINS = 10
_ALPHA = 0.75
_MOMENTUM = 0.9


def _ghm_stream_kernel(x_ref, t_ref, out_ref):
    x = x_ref[...]
    t = t_ref[...]
    br, c = x.shape

    col = lax.broadcasted_iota(jnp.int32, (br, c), 1)
    ohm = col == t

    m1 = jnp.max(x, axis=1, keepdims=True)
    e1 = jnp.exp(x - m1)
    s1 = jnp.sum(e1, axis=1, keepdims=True)
    p = e1 / s1

    g = jnp.where(ohm, 1.0 - p, p)

    s2 = jnp.sum(jnp.exp(p), axis=1, keepdims=True)
    pt = jnp.sum(jnp.where(ohm, p, 0.0), axis=1, keepdims=True)
    ce = jnp.log(s2) - pt
    ce2 = jnp.broadcast_to(ce, (br, c))

    bif = jnp.minimum(jnp.floor(g * 10.0), 9.0)

    U = [jnp.float32(0.0)] * (_NBINS + 1)
    T = [jnp.float32(0.0)] * (_NBINS + 1)
    mk = bif
    for k in range(_NBINS - 1, 0, -1):
        U[k] = jnp.sum(mk)
        T[k] = jnp.sum(mk * ce2)
        if k > 1:
            mk = jnp.minimum(mk, np.float32(k - 1))

    cum_c = [None] * (_NBINS + 1)
    cum_s = [None] * (_NBINS + 1)
    for k in range(1, _NBINS):
        cum_c[k] = U[k] - U[k - 1]
        cum_s[k] = T[k] - T[k - 1]
    cum_c[0] = jnp.float32(br * c)
    cum_s[0] = c * jnp.sum(ce)
    cum_c[_NBINS] = jnp.float32(0.0)
    cum_s[_NBINS] = jnp.float32(0.0)

    lane = lax.broadcasted_iota(jnp.int32, (1, 1, 128), 2)
    vec = jnp.zeros((1, 1, 128), jnp.float32)
    for b in range(_NBINS):
        vec = jnp.where(lane == b, cum_c[b] - cum_c[b + 1], vec)
        vec = jnp.where(lane == 16 + b, cum_s[b] - cum_s[b + 1], vec)
    out_ref[...] = vec


def _ghm_final_kernel(rows_ref, loss_ref, *, total):
    rows = rows_ref[...]
    v = jnp.sum(rows, axis=0)
    lane = lax.broadcasted_iota(jnp.int32, (1, 128), 1)
    cnt_vec = v
    s_vec = jnp.zeros((1, 128), jnp.float32)
    for b in range(_NBINS):
        s_vec = jnp.where(lane == b, jnp.sum(jnp.where(lane == 16 + b, v, 0.0)), s_vec)
    lane_ok = lane < _NBINS
    validf = jnp.where(lane_ok & (cnt_vec > 0), 1.0, 0.0)
    n = jnp.sum(validf)
    acc = jnp.maximum((1.0 - _MOMENTUM) * cnt_vec, 1e-12)
    w = jnp.exp(_ALPHA * jnp.log(total / (n * acc)))
    w = w * validf
    loss = jnp.sum(w * s_vec) / total
    loss_ref[...] = jnp.full_like(loss_ref, loss)


@functools.partial(jax.jit, static_argnames=("block_rows", "interpret"))
def _run(pred, target, block_rows=1024, interpret=False):
    n, c = pred.shape
    nblocks = n // block_rows
    t2 = target.reshape(n, 1).astype(jnp.int32)
    rows = pl.pallas_call(
        _ghm_stream_kernel,
        grid=(nblocks,),
        in_specs=[
            pl.BlockSpec((block_rows, c), lambda i: (i, 0)),
            pl.BlockSpec((block_rows, 1), lambda i: (i, 0)),
        ],
        out_specs=pl.BlockSpec((1, 1, 128), lambda i: (i, 0, 0)),
        out_shape=jax.ShapeDtypeStruct((nblocks, 1, 128), jnp.float32),
        compiler_params=pltpu.CompilerParams(
            dimension_semantics=("parallel",)),
        interpret=interpret,
    )(pred, t2)
    loss = pl.pallas_call(
        functools.partial(_ghm_final_kernel, total=float(n * c)),
        in_specs=[pl.BlockSpec((nblocks, 1, 128), lambda: (0, 0, 0))],
        out_specs=pl.BlockSpec((1, 128), lambda: (0, 0)),
        out_shape=jax.ShapeDtypeStruct((1, 128), jnp.float32),
        interpret=interpret,
    )(rows)
    return loss[0, 0]


def kernel(pred, target):
    return _run(pred, target)


# speedup vs baseline: 1.0777x; 1.0777x over previous
import functools

import numpy as np
import jax
import jax.numpy as jnp
from jax import lax
from jax.experimental import pallas as pl
from jax.experimental.pallas import tpu as pltpu

_NBINS = 10
_ALPHA = 0.75
_MOMENTUM = 0.9


def _ghm_stream_kernel(x_ref, t_ref, out_ref):
    x = x_ref[...]
    t = t_ref[...]
    br, c = x.shape

    col = lax.broadcasted_iota(jnp.int32, (br, c), 1)
    ohm = col == t

    m1 = jnp.max(x, axis=1, keepdims=True)
    e1 = jnp.exp(x - m1)
    s1 = jnp.sum(e1, axis=1, keepdims=True)
    p = e1 / s1

    g = jnp.where(ohm, 1.0 - p, p)

    s2 = jnp.sum(jnp.exp(p), axis=1, keepdims=True)
    pt = jnp.sum(jnp.where(ohm, p, 0.0), axis=1, keepdims=True)
    ce = jnp.log(s2) - pt
    ce2 = jnp.broadcast_to(ce, (br, c))

    bif = jnp.minimum(jnp.floor(g * 10.0), 9.0)

    U = [jnp.float32(0.0)] * (_NBINS + 1)
    T = [jnp.float32(0.0)] * (_NBINS + 1)
    mk = bif
    for k in range(_NBINS - 1, 0, -1):
        U[k] = jnp.sum(mk)
        T[k] = jnp.sum(mk * ce2)
        if k > 1:
            mk = jnp.minimum(mk, np.float32(k - 1))

    cum_c = [None] * (_NBINS + 1)
    cum_s = [None] * (_NBINS + 1)
    for k in range(1, _NBINS):
        cum_c[k] = U[k] - U[k - 1]
        cum_s[k] = T[k] - T[k - 1]
    cum_c[0] = jnp.float32(br * c)
    cum_s[0] = c * jnp.sum(ce)
    cum_c[_NBINS] = jnp.float32(0.0)
    cum_s[_NBINS] = jnp.float32(0.0)

    lane = lax.broadcasted_iota(jnp.int32, (1, 1, 128), 2)
    vec = jnp.zeros((1, 1, 128), jnp.float32)
    for b in range(_NBINS):
        vec = jnp.where(lane == b, cum_c[b] - cum_c[b + 1], vec)
        vec = jnp.where(lane == 16 + b, cum_s[b] - cum_s[b + 1], vec)
    out_ref[...] = vec


def _ghm_final_kernel(rows_ref, loss_ref, *, total):
    rows = rows_ref[...]
    v = jnp.sum(rows, axis=0)
    lane = lax.broadcasted_iota(jnp.int32, (1, 128), 1)
    cnt_vec = v
    s_vec = jnp.zeros((1, 128), jnp.float32)
    for b in range(_NBINS):
        s_vec = jnp.where(lane == b, jnp.sum(jnp.where(lane == 16 + b, v, 0.0)), s_vec)
    lane_ok = lane < _NBINS
    validf = jnp.where(lane_ok & (cnt_vec > 0), 1.0, 0.0)
    n = jnp.sum(validf)
    acc = jnp.maximum((1.0 - _MOMENTUM) * cnt_vec, 1e-12)
    w = jnp.exp(_ALPHA * jnp.log(total / (n * acc)))
    w = w * validf
    loss = jnp.sum(w * s_vec) / total
    loss_ref[...] = jnp.full_like(loss_ref, loss)


@functools.partial(jax.jit, static_argnames=("block_rows", "interpret"))
def _run(pred, target, block_rows=4096, interpret=False):
    n, c = pred.shape
    nblocks = n // block_rows
    t2 = target.reshape(n, 1).astype(jnp.int32)
    rows = pl.pallas_call(
        _ghm_stream_kernel,
        grid=(nblocks,),
        in_specs=[
            pl.BlockSpec((block_rows, c), lambda i: (i, 0)),
            pl.BlockSpec((block_rows, 1), lambda i: (i, 0)),
        ],
        out_specs=pl.BlockSpec((1, 1, 128), lambda i: (i, 0, 0)),
        out_shape=jax.ShapeDtypeStruct((nblocks, 1, 128), jnp.float32),
        compiler_params=pltpu.CompilerParams(
            dimension_semantics=("parallel",),
            vmem_limit_bytes=128 * 1024 * 1024),
        interpret=interpret,
    )(pred, t2)
    loss = pl.pallas_call(
        functools.partial(_ghm_final_kernel, total=float(n * c)),
        in_specs=[pl.BlockSpec((nblocks, 1, 128), lambda: (0, 0, 0))],
        out_specs=pl.BlockSpec((1, 128), lambda: (0, 0)),
        out_shape=jax.ShapeDtypeStruct((1, 128), jnp.float32),
        interpret=interpret,
    )(rows)
    return loss[0, 0]


def kernel(pred, target):
    return _run(pred, target)
